# baseline (speedup 1.0000x reference)
import functools

import jax
import jax.numpy as jnp
from jax import lax
from jax.experimental import pallas as pl
from jax.experimental.pallas import tpu as pltpu
from jax.experimental.pallas import tpu_sc as plsc

VOCAB_SIZE = 1000
DEPTH = VOCAB_SIZE + 1
MASK_TOKEN = -1
PAD_TOKEN = -2

BATCH = 1024
SEQ = 50
ROWS = BATCH * SEQ
NUM_CORES = 2
NUM_SUBCORES = 16
NW = NUM_CORES * NUM_SUBCORES
ROWS_PER_W = ROWS // NW
LANES = 16
GROUPS = ROWS_PER_W // LANES
GSZ = LANES * DEPTH
NBUF = 4


def _ohe_body(x_hbm, m_hbm, z_hbm, out_hbm, xv, mv, bufs, sems):
    wid = lax.axis_index("s") * NUM_CORES + lax.axis_index("c")
    base = wid * ROWS_PER_W

    pltpu.sync_copy(x_hbm.at[pl.ds(base * 1, ROWS_PER_W)], xv)
    pltpu.sync_copy(m_hbm.at[pl.ds(base * 1, ROWS_PER_W)], mv)

    lane = lax.broadcasted_iota(jnp.int32, (LANES,), 0)
    zeros16 = jnp.zeros((LANES,), jnp.float32)

    def flat_idx(g):
        xi = xv[pl.ds(g * LANES, LANES)]
        xi = jnp.where(xi == PAD_TOKEN, VOCAB_SIZE, xi)
        return lane * DEPTH + xi

    def keep_vals(g):
        m = mv[pl.ds(g * LANES, LANES)]
        bad = (m == float(PAD_TOKEN)) | (m == float(MASK_TOKEN))
        return jnp.where(bad, 0.0, 1.0).astype(jnp.float32)

    def out_slice(g):
        return out_hbm.at[pl.ds((base + g * LANES) * DEPTH, GSZ)]

    for b in range(NBUF):
        pltpu.sync_copy(z_hbm, bufs[b])
    for b in range(NBUF):
        plsc.store_scatter(bufs[b], [flat_idx(b)], keep_vals(b))
        pltpu.make_async_copy(bufs[b], out_slice(b), sems[b]).start()

    def chunk(c, carry):
        for b in range(NBUF):
            g = c * NBUF + b
            gp = g - NBUF
            pltpu.make_async_copy(bufs[b], out_slice(gp), sems[b]).wait()
            plsc.store_scatter(bufs[b], [flat_idx(gp)], zeros16)
            plsc.store_scatter(bufs[b], [flat_idx(g)], keep_vals(g))
            pltpu.make_async_copy(bufs[b], out_slice(g), sems[b]).start()
        return carry

    lax.fori_loop(1, GROUPS // NBUF, chunk, 0, unroll=False)

    for b in range(NBUF):
        g = GROUPS - NBUF + b
        pltpu.make_async_copy(bufs[b], out_slice(g), sems[b]).wait()


@jax.jit
def _masked_ohe(x, mask):
    zeros_blk = jnp.zeros((GSZ,), jnp.float32)
    xf = x.reshape(ROWS)
    mf = mask.reshape(ROWS)

    mesh = plsc.VectorSubcoreMesh(core_axis_name="c", subcore_axis_name="s")
    out_flat = pl.kernel(
        _ohe_body,
        out_type=jax.ShapeDtypeStruct((ROWS * DEPTH,), jnp.float32),
        mesh=mesh,
        scratch_types=[
            pltpu.VMEM((ROWS_PER_W,), jnp.int32),
            pltpu.VMEM((ROWS_PER_W,), jnp.float32),
            [pltpu.VMEM((GSZ,), jnp.float32) for _ in range(NBUF)],
            [pltpu.SemaphoreType.DMA for _ in range(NBUF)],
        ],
        compiler_params=pltpu.CompilerParams(needs_layout_passes=False),
    )(xf, mf, zeros_blk)
    return out_flat.reshape(BATCH, SEQ, DEPTH)


def kernel(x, mask):
    return _masked_ohe(x.astype(jnp.int32), mask.astype(jnp.float32))


# speedup vs baseline: 2.5155x; 2.5155x over previous
import functools

import jax
import jax.numpy as jnp
from jax import lax
from jax.experimental import pallas as pl
from jax.experimental.pallas import tpu as pltpu
from jax.experimental.pallas import tpu_sc as plsc

VOCAB_SIZE = 1000
DEPTH = VOCAB_SIZE + 1
MASK_TOKEN = -1
PAD_TOKEN = -2

BATCH = 1024
SEQ = 50
ROWS = BATCH * SEQ
NUM_CORES = 2
NUM_SUBCORES = 16
NW = NUM_CORES * NUM_SUBCORES
ROWS_PER_W = ROWS // NW
LANES = 16
GROUPS = ROWS_PER_W // LANES
GSZ = LANES * DEPTH
NBUF = 4


def _ohe_body(x_hbm, m_hbm, z_hbm, out_hbm, xv, mv, bufs, sems):
    wid = lax.axis_index("s") * NUM_CORES + lax.axis_index("c")
    base = wid * ROWS_PER_W

    pltpu.sync_copy(x_hbm.at[pl.ds(base * 1, ROWS_PER_W)], xv)
    pltpu.sync_copy(m_hbm.at[pl.ds(base * 1, ROWS_PER_W)], mv)

    lane = lax.broadcasted_iota(jnp.int32, (LANES,), 0)
    zeros16 = jnp.zeros((LANES,), jnp.float32)

    def col_idx(g):
        xi = xv[pl.ds(g * LANES, LANES)]
        return jnp.where(xi == PAD_TOKEN, VOCAB_SIZE, xi)

    def keep_vals(g):
        m = mv[pl.ds(g * LANES, LANES)]
        bad = (m == float(PAD_TOKEN)) | (m == float(MASK_TOKEN))
        return jnp.where(bad, 0.0, 1.0).astype(jnp.float32)

    def out_slice(g):
        return out_hbm.at[pl.ds(base + g * LANES, LANES)]

    for b in range(NBUF):
        pltpu.sync_copy(z_hbm, bufs[b])
    for b in range(NBUF):
        plsc.store_scatter(bufs[b], [lane, col_idx(b)], keep_vals(b))
        pltpu.make_async_copy(bufs[b], out_slice(b), sems[b]).start()

    def chunk(c, carry):
        for b in range(NBUF):
            g = c * NBUF + b
            gp = g - NBUF
            pltpu.make_async_copy(bufs[b], out_slice(gp), sems[b]).wait()
            plsc.store_scatter(bufs[b], [lane, col_idx(gp)], zeros16)
            plsc.store_scatter(bufs[b], [lane, col_idx(g)], keep_vals(g))
            pltpu.make_async_copy(bufs[b], out_slice(g), sems[b]).start()
        return carry

    lax.fori_loop(1, GROUPS // NBUF, chunk, 0, unroll=False)

    for b in range(NBUF):
        g = GROUPS - NBUF + b
        pltpu.make_async_copy(bufs[b], out_slice(g), sems[b]).wait()


@jax.jit
def _masked_ohe(x, mask):
    zeros_blk = jnp.zeros((LANES, DEPTH), jnp.float32)
    xf = x.reshape(ROWS)
    mf = mask.reshape(ROWS)

    mesh = plsc.VectorSubcoreMesh(core_axis_name="c", subcore_axis_name="s")
    out_flat = pl.kernel(
        _ohe_body,
        out_type=jax.ShapeDtypeStruct((ROWS, DEPTH), jnp.float32),
        mesh=mesh,
        scratch_types=[
            pltpu.VMEM((ROWS_PER_W,), jnp.int32),
            pltpu.VMEM((ROWS_PER_W,), jnp.float32),
            [pltpu.VMEM((LANES, DEPTH), jnp.float32) for _ in range(NBUF)],
            [pltpu.SemaphoreType.DMA for _ in range(NBUF)],
        ],
        compiler_params=pltpu.CompilerParams(needs_layout_passes=False),
    )(xf, mf, zeros_blk)
    return out_flat.reshape(BATCH, SEQ, DEPTH)


def kernel(x, mask):
    return _masked_ohe(x.astype(jnp.int32), mask.astype(jnp.float32))
